# baseline (speedup 1.0000x reference)
import functools

import jax
import jax.numpy as jnp
from jax import lax
from jax.experimental import pallas as pl
from jax.experimental.pallas import tpu as pltpu
from jax.experimental.pallas import tpu_sc as plsc

F32 = jnp.float32
I32 = jnp.int32
HI = jax.lax.Precision.HIGHEST

_N = 50000
_E = 1600000
_EL = 400000
_H = 20

_NC, _NS = 2, 16
_NW = _NC * _NS

_EPAD = 1638400
_ROWS_W = _EPAD // 128 // _NW
_CH_ROWS = 4
_NCHUNK = _ROWS_W // _CH_ROWS
_NACC = 50176
_NZR = _NACC // _NS

_ELPAD = 409600
_GW = _ELPAD // 128 // _NW


def _node_body(x_ref, nt_ref, w_ref, nb_ref, a_ref, hp_ref, wi_ref, wj_ref):
    xb = x_ref[...]
    nt = nt_ref[...]
    h = jnp.zeros((xb.shape[0], 32), F32)
    for t in range(3):
        wt = w_ref[t * 32:t * 32 + 20, :]
        m = (nt == t).astype(F32)
        h = h + m * (jax.lax.dot(xb, wt, precision=HI) + nb_ref[t:t + 1, :])
    wiwj = jax.lax.dot(h, a_ref[...], precision=HI)
    col = lax.broadcasted_iota(I32, (1, 32), 1)
    hp_ref[...] = h + (col == 20).astype(F32)
    wi_ref[...] = wiwj[:, 0:1]
    wj_ref[...] = wiwj[:, 1:2]


def _node_pass(x, nt2, wn, nbp, att_a):
    bn = 1000
    return pl.pallas_call(
        _node_body,
        grid=(_N // bn,),
        in_specs=[
            pl.BlockSpec((bn, 20), lambda i: (i, 0)),
            pl.BlockSpec((bn, 1), lambda i: (i, 0)),
            pl.BlockSpec((96, 32), lambda i: (0, 0)),
            pl.BlockSpec((8, 32), lambda i: (0, 0)),
            pl.BlockSpec((32, 8), lambda i: (0, 0)),
        ],
        out_specs=[
            pl.BlockSpec((bn, 32), lambda i: (i, 0)),
            pl.BlockSpec((bn, 1), lambda i: (i, 0)),
            pl.BlockSpec((bn, 1), lambda i: (i, 0)),
        ],
        out_shape=[
            jax.ShapeDtypeStruct((_N, 32), F32),
            jax.ShapeDtypeStruct((_N, 1), F32),
            jax.ShapeDtypeStruct((_N, 1), F32),
        ],
    )(x, nt2, wn, nbp, att_a)


def _we_body(ea_ref, et_ref, v_ref, we_ref):
    ea = ea_ref[...]
    et = et_ref[0]
    eaT = lax.transpose(ea, (1, 0))
    tvec = lax.broadcasted_iota(I32, (8, 1), 0)
    oh = (et == tvec).astype(F32)
    vsel = jax.lax.dot(v_ref[...], oh, precision=HI)
    m = eaT * vsel[:20, :]
    weT = jax.lax.dot(jnp.ones((1, 20), F32), m, precision=HI)
    we_ref[0] = weT + vsel[31:32, :]


def _we_pass(edge_attr, et3, vtp):
    be = 2000
    return pl.pallas_call(
        _we_body,
        grid=(_E // be,),
        in_specs=[
            pl.BlockSpec((be, 20), lambda i: (i, 0)),
            pl.BlockSpec((1, 1, be), lambda i: (i, 0, 0)),
            pl.BlockSpec((32, 8), lambda i: (0, 0)),
        ],
        out_specs=pl.BlockSpec((1, 1, be), lambda i: (i, 0, 0)),
        out_shape=jax.ShapeDtypeStruct((_E // be, 1, be), F32),
    )(edge_attr, et3, vtp)


def _edge_body(srcm, dstm, wem, wim, wjm, hpm, outm,
               accm, idx_s, idx_d, we_v, wig, wjg, hrows, zblk, sem):
    cid = lax.axis_index("c")
    sid = lax.axis_index("s")
    wid = cid * _NS + sid

    @pl.loop(0, 56)
    def _zfill(i):
        zblk[i, 0:16] = jnp.zeros((16,), F32)
        zblk[i, 16:32] = jnp.zeros((16,), F32)

    @pl.loop(0, _NZR // 56)
    def _zcopy(j):
        pltpu.sync_copy(zblk, accm.at[pl.ds(sid * _NZR + j * 56, 56)])

    plsc.subcore_barrier()

    @pl.loop(0, _NCHUNK)
    def _chunk(k):
        base = wid * _ROWS_W + k * _CH_ROWS
        pltpu.sync_copy(srcm.at[pl.ds(base, _CH_ROWS)], idx_s)
        pltpu.sync_copy(dstm.at[pl.ds(base, _CH_ROWS)], idx_d)
        pltpu.sync_copy(wem.at[pl.ds(base, _CH_ROWS)], we_v)
        cps = []
        for j in range(_CH_ROWS):
            cps.append(pltpu.async_copy(wim.at[idx_d.at[j]], wig.at[j], sem))
            cps.append(pltpu.async_copy(wjm.at[idx_s.at[j]], wjg.at[j], sem))
            cps.append(pltpu.async_copy(
                hpm.at[idx_s.at[j]], hrows.at[pl.ds(j * 128, 128)], sem))
        for c in cps:
            c.wait()

        @pl.loop(0, _CH_ROWS * 8)
        def _escale(i):
            j = i // 8
            off = (i % 8) * 16
            al = (we_v[j, pl.ds(off, 16)] + wig[j, pl.ds(off, 16)]
                  + wjg[j, pl.ds(off, 16)])
            al = jnp.maximum(al, 0.2 * al)
            e16 = jnp.exp(al)
            for t in range(16):
                r = i * 16 + t
                ev = jnp.full((16,), e16[t])
                hrows[r, 0:16] = hrows[r, 0:16] * ev
                hrows[r, 16:32] = hrows[r, 16:32] * ev

        for j in range(_CH_ROWS):
            pltpu.sync_copy(hrows.at[pl.ds(j * 128, 128)],
                            accm.at[idx_d.at[j]], add=True)

    plsc.subcore_barrier()
    pltpu.sync_copy(accm.at[pl.ds(sid * _NZR, _NZR)],
                    outm.at[cid].at[pl.ds(sid * _NZR, _NZR)])


def _edge_pass(srcp, dstp, wep, wi, wj, hp):
    f = functools.partial(
        pl.kernel,
        out_type=jax.ShapeDtypeStruct((_NC, _NACC, 32), F32),
        mesh=plsc.VectorSubcoreMesh(core_axis_name="c", subcore_axis_name="s"),
        compiler_params=pltpu.CompilerParams(use_tc_tiling_on_sc=False),
        scratch_types=[
            pltpu.VMEM_SHARED((_NACC, 32), F32),
            pltpu.VMEM((_CH_ROWS, 128), I32),
            pltpu.VMEM((_CH_ROWS, 128), I32),
            pltpu.VMEM((_CH_ROWS, 128), F32),
            pltpu.VMEM((_CH_ROWS, 128), F32),
            pltpu.VMEM((_CH_ROWS, 128), F32),
            pltpu.VMEM((_CH_ROWS * 128, 32), F32),
            pltpu.VMEM((56, 32), F32),
            pltpu.SemaphoreType.DMA,
        ],
    )(_edge_body)
    return f(srcp, dstp, wep, wi, wj, hp)


def _norm_body(a_ref, m_ref, z_ref):
    a = a_ref[0] + a_ref[1]
    sbro = jax.lax.dot(a, m_ref[...], precision=HI)
    z_ref[...] = a / (sbro + 1e-16)


def _norm_pass(accq, selm):
    bq = 896
    nq = _NACC * 32 // 128
    return pl.pallas_call(
        _norm_body,
        grid=(nq // bq,),
        in_specs=[
            pl.BlockSpec((_NC, bq, 128), lambda i: (0, i, 0)),
            pl.BlockSpec((128, 128), lambda i: (0, 0)),
        ],
        out_specs=pl.BlockSpec((bq, 128), lambda i: (i, 0)),
        out_shape=jax.ShapeDtypeStruct((nq, 128), F32),
    )(accq, selm)


def _dgather_body(rowm, colm, zm, outm, idxb, g0, g1, sem):
    cid = lax.axis_index("c")
    sid = lax.axis_index("s")
    wid = cid * _NS + sid

    @pl.loop(0, _GW)
    def _k(k):
        rb = wid * _GW + k
        pltpu.sync_copy(rowm.at[pl.ds(rb, 1)], idxb.at[pl.ds(0, 1)])
        pltpu.sync_copy(colm.at[pl.ds(rb, 1)], idxb.at[pl.ds(1, 1)])
        c0 = pltpu.async_copy(zm.at[idxb.at[0]], g0, sem)
        c1 = pltpu.async_copy(zm.at[idxb.at[1]], g1, sem)
        c0.wait()
        c1.wait()
        pltpu.sync_copy(g0, outm.at[0].at[pl.ds(rb * 128, 128)])
        pltpu.sync_copy(g1, outm.at[1].at[pl.ds(rb * 128, 128)])


def _dgather_pass(rowp, colp, z32):
    f = functools.partial(
        pl.kernel,
        out_type=jax.ShapeDtypeStruct((2, _ELPAD, 32), F32),
        mesh=plsc.VectorSubcoreMesh(core_axis_name="c", subcore_axis_name="s"),
        compiler_params=pltpu.CompilerParams(use_tc_tiling_on_sc=False),
        scratch_types=[
            pltpu.VMEM((2, 128), I32),
            pltpu.VMEM((128, 32), F32),
            pltpu.VMEM((128, 32), F32),
            pltpu.SemaphoreType.DMA,
        ],
    )(_dgather_body)
    return f(rowp, colp, z32)


def _dec_body(g_ref, w1_ref, b1_ref, w2_ref, o_ref):
    preds = []
    for k in range(4):
        g0k = g_ref[0][:, k * 32:(k + 1) * 32]
        g1k = g_ref[1][:, k * 32:(k + 1) * 32]
        zz = (jax.lax.dot(g0k, w1_ref[0:32, :], precision=HI)
              + jax.lax.dot(g1k, w1_ref[32:64, :], precision=HI)
              + b1_ref[0:1, :])
        zz = jnp.maximum(zz, 0.0)
        preds.append(jax.lax.dot(zz, w2_ref[...], precision=HI))
    o_ref[...] = jnp.concatenate(preds, axis=1)


def _dec_pass(gq, w1ab, b1p, w2b):
    bq = 800
    nq = _ELPAD // 4
    return pl.pallas_call(
        _dec_body,
        grid=(nq // bq,),
        in_specs=[
            pl.BlockSpec((2, bq, 128), lambda i: (0, i, 0)),
            pl.BlockSpec((64, 32), lambda i: (0, 0)),
            pl.BlockSpec((8, 32), lambda i: (0, 0)),
            pl.BlockSpec((32, 32), lambda i: (0, 0)),
        ],
        out_specs=pl.BlockSpec((bq, 128), lambda i: (i, 0)),
        out_shape=jax.ShapeDtypeStruct((nq, 128), F32),
    )(gq, w1ab, b1p, w2b)


def kernel(x, edge_index, node_type, edge_attr, edge_type, edge_label_index,
           node_W, node_b, edge_W, edge_b, att_W, att_b,
           dec1_W, dec1_b, dec2_W, dec2_b):
    h = _H
    a1 = att_W[:h, 0]
    a2 = att_W[h:2 * h, 0]
    a3 = att_W[2 * h:, 0]
    att_a = jnp.zeros((32, 8), F32).at[:h, 0].set(a1).at[:h, 1].set(a2)
    wn = jnp.zeros((3, 32, 32), F32).at[:, :h, :h].set(node_W).reshape(96, 32)
    nbp = jnp.zeros((8, 32), F32).at[:3, :h].set(node_b)
    v = jnp.einsum('tdh,h->td', edge_W, a3)
    c = edge_b @ a3 + att_b[0]
    vtp = (jnp.zeros((32, 8), F32).at[:h, :5].set(v.T).at[31, :5].set(c))

    nt2 = node_type.reshape(_N, 1).astype(I32)
    et3 = edge_type.reshape(-1, 1, 2000).astype(I32)

    hp, wi2, wj2 = _node_pass(x, nt2, wn, nbp, att_a)
    wi = wi2.reshape(_N)
    wj = wj2.reshape(_N)
    we = _we_pass(edge_attr, et3, vtp)

    pad = _EPAD - _E
    spread = (jnp.arange(pad, dtype=I32) * 37) % _N
    srcp = jnp.concatenate(
        [edge_index[0].astype(I32), spread]).reshape(-1, 128)
    dstp = jnp.concatenate(
        [edge_index[1].astype(I32), spread]).reshape(-1, 128)
    wep = jnp.concatenate(
        [we.reshape(_E), jnp.full((pad,), -1e30, F32)]).reshape(-1, 128)

    acc = _edge_pass(srcp, dstp, wep, wi, wj, hp)
    accq = acc.reshape(_NC, _NACC * 32 // 128, 128)
    rowi = lax.broadcasted_iota(I32, (128, 128), 0)
    coli = lax.broadcasted_iota(I32, (128, 128), 1)
    selm = (rowi == 20 + 32 * (coli // 32)).astype(F32)
    z32q = _norm_pass(accq, selm)
    z32 = z32q.reshape(_NACC, 32)

    lpad = _ELPAD - _EL
    rowp = jnp.concatenate(
        [edge_label_index[0].astype(I32), jnp.zeros((lpad,), I32)]
    ).reshape(-1, 128)
    colp = jnp.concatenate(
        [edge_label_index[1].astype(I32), jnp.zeros((lpad,), I32)]
    ).reshape(-1, 128)
    g = _dgather_pass(rowp, colp, z32)

    gq = g.reshape(2, _ELPAD // 4, 128)
    w1ab = (jnp.zeros((64, 32), F32)
            .at[0:h, 0:h].set(dec1_W[:h])
            .at[32:32 + h, 0:h].set(dec1_W[h:]))
    b1p = jnp.zeros((8, 32), F32).at[0, :h].set(dec1_b).at[0, 31].set(1.0)
    w2b = (jnp.zeros((32, 32), F32)
           .at[:h, :].set(jnp.broadcast_to(dec2_W[:, 0:1], (h, 32)))
           .at[31, :].set(dec2_b[0]))
    predq = _dec_pass(gq, w1ab, b1p, w2b)

    pred = lax.slice(predq, (0, 0), (_ELPAD // 4, 128), (1, 32))
    pred = pred.reshape(_ELPAD)[:_EL]
    z = z32[:_N, :h]
    return (pred, z)


# speedup vs baseline: 1.0962x; 1.0962x over previous
import functools

import jax
import jax.numpy as jnp
from jax import lax
from jax.experimental import pallas as pl
from jax.experimental.pallas import tpu as pltpu
from jax.experimental.pallas import tpu_sc as plsc

F32 = jnp.float32
I32 = jnp.int32
HI = jax.lax.Precision.HIGHEST

_N = 50000
_E = 1600000
_EL = 400000
_H = 20

_NC, _NS = 2, 16
_NW = _NC * _NS

_EPAD = 1638400
_ROWS_W = _EPAD // 128 // _NW
_CH_ROWS = 4
_NCHUNK = _ROWS_W // _CH_ROWS
_NACC = 50176
_NZR = _NACC // _NS

_ELPAD = 409600
_GW = _ELPAD // 128 // _NW


def _node_body(x_ref, nt_ref, w_ref, nb_ref, a_ref, hp_ref, wi_ref, wj_ref):
    xb = x_ref[...]
    nt = nt_ref[...]
    h = jnp.zeros((xb.shape[0], 32), F32)
    for t in range(3):
        wt = w_ref[t * 32:t * 32 + 20, :]
        m = (nt == t).astype(F32)
        h = h + m * (jax.lax.dot(xb, wt, precision=HI) + nb_ref[t:t + 1, :])
    wiwj = jax.lax.dot(h, a_ref[...], precision=HI)
    col = lax.broadcasted_iota(I32, (1, 32), 1)
    hp_ref[...] = h + (col == 20).astype(F32)
    wi_ref[...] = wiwj[:, 0:1]
    wj_ref[...] = wiwj[:, 1:2]


def _node_pass(x, nt2, wn, nbp, att_a):
    bn = 1000
    return pl.pallas_call(
        _node_body,
        grid=(_N // bn,),
        in_specs=[
            pl.BlockSpec((bn, 20), lambda i: (i, 0)),
            pl.BlockSpec((bn, 1), lambda i: (i, 0)),
            pl.BlockSpec((96, 32), lambda i: (0, 0)),
            pl.BlockSpec((8, 32), lambda i: (0, 0)),
            pl.BlockSpec((32, 8), lambda i: (0, 0)),
        ],
        out_specs=[
            pl.BlockSpec((bn, 32), lambda i: (i, 0)),
            pl.BlockSpec((bn, 1), lambda i: (i, 0)),
            pl.BlockSpec((bn, 1), lambda i: (i, 0)),
        ],
        out_shape=[
            jax.ShapeDtypeStruct((_N, 32), F32),
            jax.ShapeDtypeStruct((_N, 1), F32),
            jax.ShapeDtypeStruct((_N, 1), F32),
        ],
    )(x, nt2, wn, nbp, att_a)


def _we_body(ea_ref, et_ref, v_ref, we_ref):
    ea = ea_ref[...]
    et = et_ref[0]
    eaT = lax.transpose(ea, (1, 0))
    tvec = lax.broadcasted_iota(I32, (8, 1), 0)
    oh = (et == tvec).astype(F32)
    vsel = jax.lax.dot(v_ref[...], oh, precision=HI)
    m = eaT * vsel[:20, :]
    weT = jax.lax.dot(jnp.ones((1, 20), F32), m, precision=HI)
    we_ref[0] = weT + vsel[31:32, :]


def _we_pass(edge_attr, et3, vtp):
    be = 2000
    return pl.pallas_call(
        _we_body,
        grid=(_E // be,),
        in_specs=[
            pl.BlockSpec((be, 20), lambda i: (i, 0)),
            pl.BlockSpec((1, 1, be), lambda i: (i, 0, 0)),
            pl.BlockSpec((32, 8), lambda i: (0, 0)),
        ],
        out_specs=pl.BlockSpec((1, 1, be), lambda i: (i, 0, 0)),
        out_shape=jax.ShapeDtypeStruct((_E // be, 1, be), F32),
    )(edge_attr, et3, vtp)


def _edge_body(srcm, dstm, wem, wim, wjm, hpm, outm,
               accm, idx_s, idx_d, we_v, wig, wjg, hrows, zblk, sem):
    cid = lax.axis_index("c")
    sid = lax.axis_index("s")
    wid = cid * _NS + sid

    @pl.loop(0, 56)
    def _zfill(i):
        zblk[i, 0:16] = jnp.zeros((16,), F32)
        zblk[i, 16:32] = jnp.zeros((16,), F32)

    @pl.loop(0, _NZR // 56)
    def _zcopy(j):
        pltpu.sync_copy(zblk, accm.at[pl.ds(sid * _NZR + j * 56, 56)])

    plsc.subcore_barrier()

    @pl.loop(0, _NCHUNK)
    def _chunk(k):
        base = wid * _ROWS_W + k * _CH_ROWS
        pltpu.sync_copy(srcm.at[pl.ds(base, _CH_ROWS)], idx_s)
        pltpu.sync_copy(dstm.at[pl.ds(base, _CH_ROWS)], idx_d)
        pltpu.sync_copy(wem.at[pl.ds(base, _CH_ROWS)], we_v)
        cps = []
        for j in range(_CH_ROWS):
            cps.append(pltpu.async_copy(wim.at[idx_d.at[j]], wig.at[j], sem))
            cps.append(pltpu.async_copy(wjm.at[idx_s.at[j]], wjg.at[j], sem))
            cps.append(pltpu.async_copy(
                hpm.at[idx_s.at[j]], hrows.at[pl.ds(j * 128, 128)], sem))
        for c in cps:
            c.wait()

        @pl.loop(0, _CH_ROWS * 8)
        def _escale(i):
            j = i // 8
            off = (i % 8) * 16
            al = (we_v[j, pl.ds(off, 16)] + wig[j, pl.ds(off, 16)]
                  + wjg[j, pl.ds(off, 16)])
            al = jnp.maximum(al, 0.2 * al)
            e16 = jnp.exp(al)
            for t in range(16):
                r = i * 16 + t
                ev = jnp.full((16,), e16[t])
                hrows[r, 0:16] = hrows[r, 0:16] * ev
                hrows[r, 16:32] = hrows[r, 16:32] * ev

        for j in range(_CH_ROWS):
            pltpu.sync_copy(hrows.at[pl.ds(j * 128, 128)],
                            accm.at[idx_d.at[j]], add=True)

    plsc.subcore_barrier()
    pltpu.sync_copy(accm.at[pl.ds(sid * _NZR, _NZR)],
                    outm.at[cid].at[pl.ds(sid * _NZR, _NZR)])


def _edge_pass(srcp, dstp, wep, wi, wj, hp):
    f = functools.partial(
        pl.kernel,
        out_type=jax.ShapeDtypeStruct((_NC, _NACC, 32), F32),
        mesh=plsc.VectorSubcoreMesh(core_axis_name="c", subcore_axis_name="s"),
        compiler_params=pltpu.CompilerParams(use_tc_tiling_on_sc=False),
        scratch_types=[
            pltpu.VMEM_SHARED((_NACC, 32), F32),
            pltpu.VMEM((_CH_ROWS, 128), I32),
            pltpu.VMEM((_CH_ROWS, 128), I32),
            pltpu.VMEM((_CH_ROWS, 128), F32),
            pltpu.VMEM((_CH_ROWS, 128), F32),
            pltpu.VMEM((_CH_ROWS, 128), F32),
            pltpu.VMEM((_CH_ROWS * 128, 32), F32),
            pltpu.VMEM((56, 32), F32),
            pltpu.SemaphoreType.DMA,
        ],
    )(_edge_body)
    return f(srcp, dstp, wep, wi, wj, hp)


def _norm_body(a_ref, m_ref, z_ref):
    a = a_ref[0] + a_ref[1]
    sbro = jax.lax.dot(a, m_ref[...], precision=HI)
    z_ref[...] = a / (sbro + 1e-16)


def _norm_pass(accq, selm):
    bq = 896
    nq = _NACC * 32 // 128
    return pl.pallas_call(
        _norm_body,
        grid=(nq // bq,),
        in_specs=[
            pl.BlockSpec((_NC, bq, 128), lambda i: (0, i, 0)),
            pl.BlockSpec((128, 128), lambda i: (0, 0)),
        ],
        out_specs=pl.BlockSpec((bq, 128), lambda i: (i, 0)),
        out_shape=jax.ShapeDtypeStruct((nq, 128), F32),
    )(accq, selm)


def _dgather_body(rowm, colm, zm, outm, idxb, g0, g1, sem):
    cid = lax.axis_index("c")
    sid = lax.axis_index("s")
    wid = cid * _NS + sid

    @pl.loop(0, _GW)
    def _k(k):
        rb = wid * _GW + k
        pltpu.sync_copy(rowm.at[pl.ds(rb, 1)], idxb.at[pl.ds(0, 1)])
        pltpu.sync_copy(colm.at[pl.ds(rb, 1)], idxb.at[pl.ds(1, 1)])
        c0 = pltpu.async_copy(zm.at[idxb.at[0]], g0, sem)
        c1 = pltpu.async_copy(zm.at[idxb.at[1]], g1, sem)
        c0.wait()
        c1.wait()
        pltpu.sync_copy(g0, outm.at[0].at[pl.ds(rb * 128, 128)])
        pltpu.sync_copy(g1, outm.at[1].at[pl.ds(rb * 128, 128)])


def _dgather_pass(rowp, colp, z32):
    f = functools.partial(
        pl.kernel,
        out_type=jax.ShapeDtypeStruct((2, _ELPAD, 32), F32),
        mesh=plsc.VectorSubcoreMesh(core_axis_name="c", subcore_axis_name="s"),
        compiler_params=pltpu.CompilerParams(use_tc_tiling_on_sc=False),
        scratch_types=[
            pltpu.VMEM((2, 128), I32),
            pltpu.VMEM((128, 32), F32),
            pltpu.VMEM((128, 32), F32),
            pltpu.SemaphoreType.DMA,
        ],
    )(_dgather_body)
    return f(rowp, colp, z32)


def _dec_body(g_ref, w1_ref, b1_ref, w2_ref, o_ref):
    g0 = g_ref[0]
    g1 = g_ref[1]
    bq = g0.shape[0]
    zs = [jnp.concatenate([g0[:, k * 32:(k + 1) * 32],
                           g1[:, k * 32:(k + 1) * 32]], axis=1)
          for k in range(4)]
    zmat = jnp.concatenate(zs, axis=0)
    zz = jnp.maximum(
        jax.lax.dot(zmat, w1_ref[...], precision=HI) + b1_ref[0:1, :], 0.0)
    p = jax.lax.dot(zz, w2_ref[...], precision=HI)
    o_ref[...] = jnp.concatenate(
        [p[k * bq:(k + 1) * bq] for k in range(4)], axis=1)


def _dec_pass(gq, w1ab, b1p, w2b):
    bq = 3200
    nq = _ELPAD // 4
    return pl.pallas_call(
        _dec_body,
        grid=(nq // bq,),
        in_specs=[
            pl.BlockSpec((2, bq, 128), lambda i: (0, i, 0)),
            pl.BlockSpec((64, 32), lambda i: (0, 0)),
            pl.BlockSpec((8, 32), lambda i: (0, 0)),
            pl.BlockSpec((32, 32), lambda i: (0, 0)),
        ],
        out_specs=pl.BlockSpec((bq, 128), lambda i: (i, 0)),
        out_shape=jax.ShapeDtypeStruct((nq, 128), F32),
    )(gq, w1ab, b1p, w2b)


def kernel(x, edge_index, node_type, edge_attr, edge_type, edge_label_index,
           node_W, node_b, edge_W, edge_b, att_W, att_b,
           dec1_W, dec1_b, dec2_W, dec2_b):
    h = _H
    a1 = att_W[:h, 0]
    a2 = att_W[h:2 * h, 0]
    a3 = att_W[2 * h:, 0]
    att_a = jnp.zeros((32, 8), F32).at[:h, 0].set(a1).at[:h, 1].set(a2)
    wn = jnp.zeros((3, 32, 32), F32).at[:, :h, :h].set(node_W).reshape(96, 32)
    nbp = jnp.zeros((8, 32), F32).at[:3, :h].set(node_b)
    v = jnp.einsum('tdh,h->td', edge_W, a3)
    c = edge_b @ a3 + att_b[0]
    vtp = (jnp.zeros((32, 8), F32).at[:h, :5].set(v.T).at[31, :5].set(c))

    nt2 = node_type.reshape(_N, 1).astype(I32)
    et3 = edge_type.reshape(-1, 1, 2000).astype(I32)

    hp, wi2, wj2 = _node_pass(x, nt2, wn, nbp, att_a)
    wi = wi2.reshape(_N)
    wj = wj2.reshape(_N)
    we = _we_pass(edge_attr, et3, vtp)

    pad = _EPAD - _E
    spread = (jnp.arange(pad, dtype=I32) * 37) % _N
    srcp = jnp.concatenate(
        [edge_index[0].astype(I32), spread]).reshape(-1, 128)
    dstp = jnp.concatenate(
        [edge_index[1].astype(I32), spread]).reshape(-1, 128)
    wep = jnp.concatenate(
        [we.reshape(_E), jnp.full((pad,), -1e30, F32)]).reshape(-1, 128)

    acc = _edge_pass(srcp, dstp, wep, wi, wj, hp)
    accq = acc.reshape(_NC, _NACC * 32 // 128, 128)
    rowi = lax.broadcasted_iota(I32, (128, 128), 0)
    coli = lax.broadcasted_iota(I32, (128, 128), 1)
    selm = (rowi == 20 + 32 * (coli // 32)).astype(F32)
    z32q = _norm_pass(accq, selm)
    z32 = z32q.reshape(_NACC, 32)

    lpad = _ELPAD - _EL
    rowp = jnp.concatenate(
        [edge_label_index[0].astype(I32), jnp.zeros((lpad,), I32)]
    ).reshape(-1, 128)
    colp = jnp.concatenate(
        [edge_label_index[1].astype(I32), jnp.zeros((lpad,), I32)]
    ).reshape(-1, 128)
    g = _dgather_pass(rowp, colp, z32)

    gq = g.reshape(2, _ELPAD // 4, 128)
    w1ab = (jnp.zeros((64, 32), F32)
            .at[0:h, 0:h].set(dec1_W[:h])
            .at[32:32 + h, 0:h].set(dec1_W[h:]))
    b1p = jnp.zeros((8, 32), F32).at[0, :h].set(dec1_b).at[0, 31].set(1.0)
    w2b = (jnp.zeros((32, 32), F32)
           .at[:h, :].set(jnp.broadcast_to(dec2_W[:, 0:1], (h, 32)))
           .at[31, :].set(dec2_b[0]))
    predq = _dec_pass(gq, w1ab, b1p, w2b)

    pred = lax.slice(predq, (0, 0), (_ELPAD // 4, 128), (1, 32))
    pred = pred.reshape(_ELPAD)[:_EL]
    z = z32[:_N, :h]
    return (pred, z)


# speedup vs baseline: 1.1373x; 1.0375x over previous
import functools

import jax
import jax.numpy as jnp
from jax import lax
from jax.experimental import pallas as pl
from jax.experimental.pallas import tpu as pltpu
from jax.experimental.pallas import tpu_sc as plsc

F32 = jnp.float32
I32 = jnp.int32
HI = jax.lax.Precision.HIGHEST

_N = 50000
_E = 1600000
_EL = 400000
_H = 20

_NC, _NS = 2, 16
_NW = _NC * _NS

_EPAD = 1638400
_ROWS_W = _EPAD // 128 // _NW
_CH_ROWS = 4
_NCHUNK = _ROWS_W // _CH_ROWS
_NACC = 50176
_NZR = _NACC // _NS

_ELPAD = 409600
_GW = _ELPAD // 128 // _NW


def _node_body(x_ref, nt_ref, w_ref, nb_ref, a_ref, hp_ref, wi_ref, wj_ref):
    xb = x_ref[...]
    nt = nt_ref[...]
    h = jnp.zeros((xb.shape[0], 32), F32)
    for t in range(3):
        wt = w_ref[t * 32:t * 32 + 20, :]
        m = (nt == t).astype(F32)
        h = h + m * (jax.lax.dot(xb, wt, precision=HI) + nb_ref[t:t + 1, :])
    wiwj = jax.lax.dot(h, a_ref[...], precision=HI)
    col = lax.broadcasted_iota(I32, (1, 32), 1)
    hp_ref[...] = h + (col == 20).astype(F32)
    wi_ref[...] = wiwj[:, 0:1]
    wj_ref[...] = wiwj[:, 1:2]


def _node_pass(x, nt2, wn, nbp, att_a):
    bn = 1000
    return pl.pallas_call(
        _node_body,
        grid=(_N // bn,),
        in_specs=[
            pl.BlockSpec((bn, 20), lambda i: (i, 0)),
            pl.BlockSpec((bn, 1), lambda i: (i, 0)),
            pl.BlockSpec((96, 32), lambda i: (0, 0)),
            pl.BlockSpec((8, 32), lambda i: (0, 0)),
            pl.BlockSpec((32, 8), lambda i: (0, 0)),
        ],
        out_specs=[
            pl.BlockSpec((bn, 32), lambda i: (i, 0)),
            pl.BlockSpec((bn, 1), lambda i: (i, 0)),
            pl.BlockSpec((bn, 1), lambda i: (i, 0)),
        ],
        out_shape=[
            jax.ShapeDtypeStruct((_N, 32), F32),
            jax.ShapeDtypeStruct((_N, 1), F32),
            jax.ShapeDtypeStruct((_N, 1), F32),
        ],
    )(x, nt2, wn, nbp, att_a)


def _we_body(ea_ref, et_ref, v_ref, we_ref):
    ea = ea_ref[...]
    et = et_ref[0]
    eaT = lax.transpose(ea, (1, 0))
    tvec = lax.broadcasted_iota(I32, (8, 1), 0)
    oh = (et == tvec).astype(F32)
    vsel = jax.lax.dot(v_ref[...], oh, precision=HI)
    m = eaT * vsel[:20, :]
    weT = jax.lax.dot(jnp.ones((1, 20), F32), m, precision=HI)
    we_ref[0] = weT + vsel[31:32, :]


def _we_pass(edge_attr, et3, vtp):
    be = 2000
    return pl.pallas_call(
        _we_body,
        grid=(_E // be,),
        in_specs=[
            pl.BlockSpec((be, 20), lambda i: (i, 0)),
            pl.BlockSpec((1, 1, be), lambda i: (i, 0, 0)),
            pl.BlockSpec((32, 8), lambda i: (0, 0)),
        ],
        out_specs=pl.BlockSpec((1, 1, be), lambda i: (i, 0, 0)),
        out_shape=jax.ShapeDtypeStruct((_E // be, 1, be), F32),
    )(edge_attr, et3, vtp)


def _edge_body(srcm, dstm, wem, wim, wjm, hpm, outm,
               accm, idx_s, idx_d, we_v, wig, wjg, hrows, zblk, sem):
    cid = lax.axis_index("c")
    sid = lax.axis_index("s")
    wid = cid * _NS + sid

    @pl.loop(0, 56)
    def _zfill(i):
        zblk[i, 0:16] = jnp.zeros((16,), F32)
        zblk[i, 16:32] = jnp.zeros((16,), F32)

    @pl.loop(0, _NZR // 56)
    def _zcopy(j):
        pltpu.sync_copy(zblk, accm.at[pl.ds(sid * _NZR + j * 56, 56)])

    plsc.subcore_barrier()

    @pl.loop(0, _NCHUNK)
    def _chunk(k):
        base = wid * _ROWS_W + k * _CH_ROWS
        pltpu.sync_copy(srcm.at[pl.ds(base, _CH_ROWS)], idx_s)
        pltpu.sync_copy(dstm.at[pl.ds(base, _CH_ROWS)], idx_d)
        pltpu.sync_copy(wem.at[pl.ds(base, _CH_ROWS)], we_v)
        cps = []
        for j in range(_CH_ROWS):
            cps.append(pltpu.async_copy(wim.at[idx_d.at[j]], wig.at[j], sem))
            cps.append(pltpu.async_copy(wjm.at[idx_s.at[j]], wjg.at[j], sem))
            cps.append(pltpu.async_copy(
                hpm.at[idx_s.at[j]], hrows.at[pl.ds(j * 128, 128)], sem))
        for c in cps:
            c.wait()

        @pl.loop(0, _CH_ROWS * 8)
        def _escale(i):
            j = i // 8
            off = (i % 8) * 16
            al = (we_v[j, pl.ds(off, 16)] + wig[j, pl.ds(off, 16)]
                  + wjg[j, pl.ds(off, 16)])
            al = jnp.maximum(al, 0.2 * al)
            e16 = jnp.exp(al)
            for t in range(16):
                r = i * 16 + t
                ev = jnp.full((16,), e16[t])
                hrows[r, 0:16] = hrows[r, 0:16] * ev
                hrows[r, 16:32] = hrows[r, 16:32] * ev

        for j in range(_CH_ROWS):
            pltpu.sync_copy(hrows.at[pl.ds(j * 128, 128)],
                            accm.at[idx_d.at[j]], add=True)

    plsc.subcore_barrier()
    pltpu.sync_copy(accm.at[pl.ds(sid * _NZR, _NZR)],
                    outm.at[cid].at[pl.ds(sid * _NZR, _NZR)])


def _edge_pass(srcp, dstp, wep, wi, wj, hp):
    f = functools.partial(
        pl.kernel,
        out_type=jax.ShapeDtypeStruct((_NC, _NACC, 32), F32),
        mesh=plsc.VectorSubcoreMesh(core_axis_name="c", subcore_axis_name="s"),
        compiler_params=pltpu.CompilerParams(use_tc_tiling_on_sc=False),
        scratch_types=[
            pltpu.VMEM_SHARED((_NACC, 32), F32),
            pltpu.VMEM((_CH_ROWS, 128), I32),
            pltpu.VMEM((_CH_ROWS, 128), I32),
            pltpu.VMEM((_CH_ROWS, 128), F32),
            pltpu.VMEM((_CH_ROWS, 128), F32),
            pltpu.VMEM((_CH_ROWS, 128), F32),
            pltpu.VMEM((_CH_ROWS * 128, 32), F32),
            pltpu.VMEM((56, 32), F32),
            pltpu.SemaphoreType.DMA,
        ],
    )(_edge_body)
    return f(srcp, dstp, wep, wi, wj, hp)


def _norm_body(a_ref, m_ref, z_ref):
    a = a_ref[0] + a_ref[1]
    sbro = jax.lax.dot(a, m_ref[...], precision=HI)
    z_ref[...] = a / (sbro + 1e-16)


def _norm_pass(accq, selm):
    bq = 896
    nq = _NACC * 32 // 128
    return pl.pallas_call(
        _norm_body,
        grid=(nq // bq,),
        in_specs=[
            pl.BlockSpec((_NC, bq, 128), lambda i: (0, i, 0)),
            pl.BlockSpec((128, 128), lambda i: (0, 0)),
        ],
        out_specs=pl.BlockSpec((bq, 128), lambda i: (i, 0)),
        out_shape=jax.ShapeDtypeStruct((nq, 128), F32),
    )(accq, selm)


def _dgather_body(rowm, colm, zm, outm, ridx, cidx, g0a, g1a, g0b, g1b,
                  sem_a, sem_b):
    cid = lax.axis_index("c")
    sid = lax.axis_index("s")
    wid = cid * _NS + sid
    base = wid * _GW
    pltpu.sync_copy(rowm.at[pl.ds(base, _GW)], ridx)
    pltpu.sync_copy(colm.at[pl.ds(base, _GW)], cidx)
    pltpu.async_copy(zm.at[ridx.at[0]], g0a, sem_a)
    pltpu.async_copy(zm.at[cidx.at[0]], g1a, sem_a)

    @pl.loop(0, _GW // 2)
    def _k(g):
        ka = 2 * g
        kb = 2 * g + 1
        pltpu.async_copy(zm.at[ridx.at[kb]], g0b, sem_b)
        pltpu.async_copy(zm.at[cidx.at[kb]], g1b, sem_b)
        pltpu.make_async_copy(zm.at[ridx.at[ka]], g0a, sem_a).wait()
        pltpu.make_async_copy(zm.at[cidx.at[ka]], g1a, sem_a).wait()
        ra = (base + ka) * 128
        pltpu.sync_copy(g0a, outm.at[0].at[pl.ds(ra, 128)])
        pltpu.sync_copy(g1a, outm.at[1].at[pl.ds(ra, 128)])

        @pl.when(g < _GW // 2 - 1)
        def _next():
            pltpu.async_copy(zm.at[ridx.at[ka + 2]], g0a, sem_a)
            pltpu.async_copy(zm.at[cidx.at[ka + 2]], g1a, sem_a)

        pltpu.make_async_copy(zm.at[ridx.at[kb]], g0b, sem_b).wait()
        pltpu.make_async_copy(zm.at[cidx.at[kb]], g1b, sem_b).wait()
        rb2 = (base + kb) * 128
        pltpu.sync_copy(g0b, outm.at[0].at[pl.ds(rb2, 128)])
        pltpu.sync_copy(g1b, outm.at[1].at[pl.ds(rb2, 128)])


def _dgather_pass(rowp, colp, z32):
    f = functools.partial(
        pl.kernel,
        out_type=jax.ShapeDtypeStruct((2, _ELPAD, 32), F32),
        mesh=plsc.VectorSubcoreMesh(core_axis_name="c", subcore_axis_name="s"),
        compiler_params=pltpu.CompilerParams(use_tc_tiling_on_sc=False),
        scratch_types=[
            pltpu.VMEM((_GW, 128), I32),
            pltpu.VMEM((_GW, 128), I32),
            pltpu.VMEM((128, 32), F32),
            pltpu.VMEM((128, 32), F32),
            pltpu.VMEM((128, 32), F32),
            pltpu.VMEM((128, 32), F32),
            pltpu.SemaphoreType.DMA,
            pltpu.SemaphoreType.DMA,
        ],
    )(_dgather_body)
    return f(rowp, colp, z32)


def _dec_body(g_ref, w1_ref, b1_ref, w2_ref, o_ref):
    g0 = g_ref[0]
    g1 = g_ref[1]
    bq = g0.shape[0]
    zs = [jnp.concatenate([g0[:, k * 32:(k + 1) * 32],
                           g1[:, k * 32:(k + 1) * 32]], axis=1)
          for k in range(4)]
    zmat = jnp.concatenate(zs, axis=0)
    zz = jnp.maximum(
        jax.lax.dot(zmat, w1_ref[...], precision=HI) + b1_ref[0:1, :], 0.0)
    p = jax.lax.dot(zz, w2_ref[...], precision=HI)
    o_ref[...] = jnp.concatenate(
        [p[k * bq:(k + 1) * bq] for k in range(4)], axis=1)


def _dec_pass(gq, w1ab, b1p, w2b):
    bq = 3200
    nq = _ELPAD // 4
    return pl.pallas_call(
        _dec_body,
        grid=(nq // bq,),
        in_specs=[
            pl.BlockSpec((2, bq, 128), lambda i: (0, i, 0)),
            pl.BlockSpec((64, 32), lambda i: (0, 0)),
            pl.BlockSpec((8, 32), lambda i: (0, 0)),
            pl.BlockSpec((32, 32), lambda i: (0, 0)),
        ],
        out_specs=pl.BlockSpec((bq, 128), lambda i: (i, 0)),
        out_shape=jax.ShapeDtypeStruct((nq, 128), F32),
    )(gq, w1ab, b1p, w2b)


def kernel(x, edge_index, node_type, edge_attr, edge_type, edge_label_index,
           node_W, node_b, edge_W, edge_b, att_W, att_b,
           dec1_W, dec1_b, dec2_W, dec2_b):
    h = _H
    a1 = att_W[:h, 0]
    a2 = att_W[h:2 * h, 0]
    a3 = att_W[2 * h:, 0]
    att_a = jnp.zeros((32, 8), F32).at[:h, 0].set(a1).at[:h, 1].set(a2)
    wn = jnp.zeros((3, 32, 32), F32).at[:, :h, :h].set(node_W).reshape(96, 32)
    nbp = jnp.zeros((8, 32), F32).at[:3, :h].set(node_b)
    v = jnp.einsum('tdh,h->td', edge_W, a3)
    c = edge_b @ a3 + att_b[0]
    vtp = (jnp.zeros((32, 8), F32).at[:h, :5].set(v.T).at[31, :5].set(c))

    nt2 = node_type.reshape(_N, 1).astype(I32)
    et3 = edge_type.reshape(-1, 1, 2000).astype(I32)

    hp, wi2, wj2 = _node_pass(x, nt2, wn, nbp, att_a)
    wi = wi2.reshape(_N)
    wj = wj2.reshape(_N)
    we = _we_pass(edge_attr, et3, vtp)

    pad = _EPAD - _E
    spread = (jnp.arange(pad, dtype=I32) * 37) % _N
    srcp = jnp.concatenate(
        [edge_index[0].astype(I32), spread]).reshape(-1, 128)
    dstp = jnp.concatenate(
        [edge_index[1].astype(I32), spread]).reshape(-1, 128)
    wep = jnp.concatenate(
        [we.reshape(_E), jnp.full((pad,), -1e30, F32)]).reshape(-1, 128)

    acc = _edge_pass(srcp, dstp, wep, wi, wj, hp)
    accq = acc.reshape(_NC, _NACC * 32 // 128, 128)
    rowi = lax.broadcasted_iota(I32, (128, 128), 0)
    coli = lax.broadcasted_iota(I32, (128, 128), 1)
    selm = (rowi == 20 + 32 * (coli // 32)).astype(F32)
    z32q = _norm_pass(accq, selm)
    z32 = z32q.reshape(_NACC, 32)

    lpad = _ELPAD - _EL
    rowp = jnp.concatenate(
        [edge_label_index[0].astype(I32), jnp.zeros((lpad,), I32)]
    ).reshape(-1, 128)
    colp = jnp.concatenate(
        [edge_label_index[1].astype(I32), jnp.zeros((lpad,), I32)]
    ).reshape(-1, 128)
    g = _dgather_pass(rowp, colp, z32)

    gq = g.reshape(2, _ELPAD // 4, 128)
    w1ab = (jnp.zeros((64, 32), F32)
            .at[0:h, 0:h].set(dec1_W[:h])
            .at[32:32 + h, 0:h].set(dec1_W[h:]))
    b1p = jnp.zeros((8, 32), F32).at[0, :h].set(dec1_b).at[0, 31].set(1.0)
    w2b = (jnp.zeros((32, 32), F32)
           .at[:h, :].set(jnp.broadcast_to(dec2_W[:, 0:1], (h, 32)))
           .at[31, :].set(dec2_b[0]))
    predq = _dec_pass(gq, w1ab, b1p, w2b)

    pred = lax.slice(predq, (0, 0), (_ELPAD // 4, 128), (1, 32))
    pred = pred.reshape(_ELPAD)[:_EL]
    z = z32[:_N, :h]
    return (pred, z)
